# baseline (speedup 1.0000x reference)
import functools

import numpy as _np

import jax
import jax.numpy as jnp
from jax import lax
from jax.experimental import pallas as pl
from jax.experimental.pallas import tpu as pltpu
from jax.experimental.pallas import tpu_sc as plsc

N = 50000
R = 8
E = 75000
D = 128

NE = R * E
NEP = 600064
EPW = NEP // 16
SEG = 4688
NSEG = EPW // SEG
NVEC = SEG // 16
K = 128
THRESH = K - 16
SBYTES = K * D * 4
NCHUNK = 8
CH = 6400
CPT = CH // 16
OUTP = NCHUNK * CH

_MESH = plsc.VectorSubcoreMesh(core_axis_name="c", subcore_axis_name="s")


@functools.partial(
    pl.kernel,
    out_type=jax.ShapeDtypeStruct((OUTP, D), jnp.float32),
    mesh=_MESH,
    compiler_params=pltpu.CompilerParams(needs_layout_passes=False,
                                         use_tc_tiling_on_sc=False),
    scratch_types=[
        pltpu.VMEM((SEG,), jnp.int32),
        pltpu.VMEM((SEG,), jnp.int32),
        pltpu.VMEM((SEG,), jnp.float32),
        pltpu.VMEM((K,), jnp.int32),
        pltpu.VMEM((K,), jnp.int32),
        pltpu.VMEM((K,), jnp.float32),
        pltpu.VMEM((K,), jnp.int32),
        pltpu.VMEM((K,), jnp.int32),
        pltpu.VMEM((K,), jnp.float32),
        pltpu.VMEM((K, D // 2), jnp.int32),
        pltpu.VMEM((K, D // 2), jnp.int32),
        pltpu.VMEM((K, D), jnp.float32),
        pltpu.VMEM((K, D), jnp.float32),
        pltpu.VMEM((K,), jnp.int32),
        pltpu.VMEM((K,), jnp.int32),
        pltpu.VMEM_SHARED((CH, D), jnp.float32),
        pltpu.SemaphoreType.DMA,
        pltpu.SemaphoreType.DMA,
        pltpu.SemaphoreType.DMA,
        pltpu.SemaphoreType.DMA,
    ],
)
def _sc_scatter(rows_hbm, cols_hbm, w_hbm, y_hbm, out_hbm,
                seg_rows, seg_cols, seg_w,
                b_cols0, b_rows0, b_w0, b_cols1, b_rows1, b_w1,
                g_buf0, g_buf1, s_buf0, s_buf1, brd0, brd1, acc,
                sem0, sem1, ssem0, ssem1):
    c = lax.axis_index("c")
    s = lax.axis_index("s")
    zero16f = jnp.zeros((16,), jnp.float32)
    zero16i = jnp.zeros((16,), jnp.int32)
    sets = ((b_cols0, b_rows0, b_w0, g_buf0, sem0, s_buf0, brd0, ssem0),
            (b_cols1, b_rows1, b_w1, g_buf1, sem1, s_buf1, brd1, ssem1))

    for i in range(K // 16):
        for (bc, br, bw, gb, sm, sb, brd, ssm) in sets:
            bw[pl.ds(i * 16, 16)] = zero16f
            bc[pl.ds(i * 16, 16)] = zero16i
            br[pl.ds(i * 16, 16)] = zero16i
            brd[pl.ds(i * 16, 16)] = zero16i

    ebase = s * EPW

    def fire(ph):
        bc, br, bw, gb, sm, sb, brd, ssm = sets[ph]
        pltpu.async_copy(y_hbm.at[bc], gb, sm)

    def process(ph):
        bc, br, bw, gb, sm, sb, brd, ssm = sets[ph]
        pltpu.make_async_copy(y_hbm.at[bc], gb, sm).wait()
        pltpu.make_async_copy(out_hbm.at[pl.ds(0, K)], sb, ssm).wait()
        for i in range(K // 16):
            brd[pl.ds(i * 16, 16)] = br[pl.ds(i * 16, 16)]

        def scale_body(k, carry):
            wspl = plsc.load_gather(bw, [zero16i + k])
            for j in range(D // 32):
                xi = gb[k, pl.ds(j * 16, 16)]
                x = plsc.bitcast(xi, jnp.bfloat16)
                a, b = plsc.unpack(x, format=plsc.PackFormat.INTERLEAVED)
                sb[k, pl.ds(j * 32, 16)] = a * wspl
                sb[k, pl.ds(j * 32 + 16, 16)] = b * wspl
            return carry

        lax.fori_loop(0, K, scale_body, 0)
        pltpu.async_copy(sb, acc.at[brd], ssm, add=True)
        for i in range(K // 16):
            bw[pl.ds(i * 16, 16)] = zero16f

    for chunk in range(NCHUNK // 2):
        lo = (c * (NCHUNK // 2) + chunk) * CH
        abase = s * CPT

        def zero_sbuf(k, carry):
            for j in range(D // 16):
                s_buf0[k, pl.ds(j * 16, 16)] = zero16f
                s_buf1[k, pl.ds(j * 16, 16)] = zero16f
            return carry

        lax.fori_loop(0, K, zero_sbuf, 0)
        nfull = CPT // K
        for t in range(nfull):
            pltpu.sync_copy(s_buf0, acc.at[pl.ds(abase + t * K, K)])
        rem = CPT - nfull * K
        if rem:
            pltpu.sync_copy(s_buf0.at[pl.ds(0, rem)],
                            acc.at[pl.ds(abase + nfull * K, rem)])
        plsc.subcore_barrier()

        pltpu.async_copy(s_buf0, acc.at[brd0], ssem0, add=True)
        pltpu.async_copy(s_buf1, acc.at[brd1], ssem1, add=True)
        fire(1)

        def seg_body(sg, carry):
            nb0, ph0 = carry
            off = ebase + sg * SEG
            pltpu.sync_copy(rows_hbm.at[pl.ds(off, SEG)], seg_rows)
            pltpu.sync_copy(cols_hbm.at[pl.ds(off, SEG)], seg_cols)
            pltpu.sync_copy(w_hbm.at[pl.ds(off, SEG)], seg_w)

            def scan_body(i, carry2):
                nb, ph = carry2
                o = i * 16
                rv = seg_rows[pl.ds(o, 16)]
                lrv = rv - lo
                m = (lrv >= 0) & (lrv < CH)
                cv = seg_cols[pl.ds(o, 16)]
                wv = seg_w[pl.ds(o, 16)]
                mi = m.astype(jnp.int32)
                incl = plsc.cumsum(mi)
                idx = (incl - mi) + nb
                for pp in range(2):
                    @pl.when(ph == pp)
                    def _():
                        bc, br, bw, gb, sm, sb, brd, ssm = sets[pp]
                        plsc.store_scatter(bc, [idx], cv, mask=m)
                        plsc.store_scatter(br, [idx], lrv, mask=m)
                        plsc.store_scatter(bw, [idx], wv, mask=m)
                nb2 = nb + jnp.max(incl)
                full = nb2 >= THRESH
                for pp in range(2):
                    @pl.when(full & (ph == pp))
                    def _():
                        fire(pp)
                        process(1 - pp)
                ph2 = jnp.where(full, 1 - ph, ph)
                nb3 = jnp.where(full, 0, nb2)
                return (nb3, ph2)

            return lax.fori_loop(0, NVEC, scan_body, (nb0, ph0))

        _, ph_end = lax.fori_loop(0, NSEG, seg_body,
                                  (jnp.int32(0), jnp.int32(0)))
        for pp in range(2):
            @pl.when(ph_end == pp)
            def _():
                fire(pp)
                process(1 - pp)
                process(pp)
        pltpu.make_async_copy(out_hbm.at[pl.ds(0, K)], s_buf0, ssem0).wait()
        pltpu.make_async_copy(out_hbm.at[pl.ds(0, K)], s_buf1, ssem1).wait()
        plsc.subcore_barrier()

        pltpu.sync_copy(acc.at[pl.ds(abase, CPT)],
                        out_hbm.at[pl.ds(lo + abase, CPT)])
        plsc.subcore_barrier()


def _w_body(rel_ref, W1_ref, b1_ref, W2t_ref, vals_ref, w_ref):
    h = jnp.tanh(jnp.dot(rel_ref[...], W1_ref[...],
                         preferred_element_type=jnp.float32) + b1_ref[...])
    logit = jnp.sum(h * W2t_ref[...], axis=1, keepdims=True)
    w_ref[...] = vals_ref[...] * jax.nn.sigmoid(logit)


def _mm_body(x_ref, w_ref, y_ref):
    y_ref[...] = jnp.dot(x_ref[...], w_ref[...],
                         preferred_element_type=jnp.float32).astype(jnp.bfloat16)


def kernel(ent_mat, rel_mat, adj_rows, adj_cols, adj_vals, weight_ent, W1, b1, W2):
    w2d = pl.pallas_call(
        _w_body,
        out_shape=jax.ShapeDtypeStruct((R, E), jnp.float32),
    )(rel_mat, W1, b1.reshape(1, D), W2.reshape(1, D), adj_vals)

    perm = _np.empty((D,), dtype=_np.int32)
    for g in range(D // 32):
        for i in range(16):
            perm[g * 32 + 2 * i] = g * 32 + i
            perm[g * 32 + 2 * i + 1] = g * 32 + 16 + i
    w_ent_p = weight_ent[:, perm]

    BM = 1000
    y = pl.pallas_call(
        _mm_body,
        grid=(N // BM,),
        in_specs=[pl.BlockSpec((BM, D), lambda i: (i, 0)),
                  pl.BlockSpec((D, D), lambda i: (0, 0))],
        out_specs=pl.BlockSpec((BM, D), lambda i: (i, 0)),
        out_shape=jax.ShapeDtypeStruct((N, D), jnp.bfloat16),
    )(ent_mat, w_ent_p)

    pad = NEP - NE
    rows_f = jnp.concatenate(
        [adj_rows.reshape(-1), jnp.full((pad,), 2 ** 30, jnp.int32)])
    cols_f = jnp.concatenate(
        [adj_cols.reshape(-1), jnp.zeros((pad,), jnp.int32)])
    w_f = jnp.concatenate([w2d.reshape(-1), jnp.zeros((pad,), jnp.float32)])

    y32 = jax.lax.bitcast_convert_type(y.reshape(N, D // 2, 2), jnp.int32)
    out_p = _sc_scatter(rows_f, cols_f, w_f, y32)
    return out_p[:N], rel_mat


# speedup vs baseline: 1.1607x; 1.1607x over previous
import functools

import numpy as _np

import jax
import jax.numpy as jnp
from jax import lax
from jax.experimental import pallas as pl
from jax.experimental.pallas import tpu as pltpu
from jax.experimental.pallas import tpu_sc as plsc

N = 50000
R = 8
E = 75000
D = 128

NE = R * E
NEP = 600064
EPW = NEP // 16
SEG = 9376
NSEG = EPW // SEG
NVEC = SEG // 16
K = 128
THRESH = K - 16
NCHUNK = 6
CH = 8448
CPT = CH // 16
OUTP = NCHUNK * CH

_MESH = plsc.VectorSubcoreMesh(core_axis_name="c", subcore_axis_name="s")


@functools.partial(
    pl.kernel,
    out_type=jax.ShapeDtypeStruct((OUTP, D), jnp.float32),
    mesh=_MESH,
    compiler_params=pltpu.CompilerParams(needs_layout_passes=False,
                                         use_tc_tiling_on_sc=False),
    scratch_types=[
        pltpu.VMEM((SEG,), jnp.int32),
        pltpu.VMEM((SEG,), jnp.int32),
        pltpu.VMEM((SEG,), jnp.float32),
        pltpu.VMEM((K,), jnp.int32),
        pltpu.VMEM((K,), jnp.int32),
        pltpu.VMEM((K,), jnp.float32),
        pltpu.VMEM((K, D // 2), jnp.int32),
        pltpu.VMEM((K, D), jnp.float32),
        pltpu.VMEM_SHARED((CH, D), jnp.float32),
        pltpu.SemaphoreType.DMA,
        pltpu.SemaphoreType.DMA,
    ],
)
def _sc_scatter(rows_hbm, cols_hbm, w_hbm, y_hbm, out_hbm,
                seg_rows, seg_cols, seg_w, b_cols, b_rows, b_w, g_buf, s_buf,
                acc, sem, sem2):
    c = lax.axis_index("c")
    s = lax.axis_index("s")
    zero16f = jnp.zeros((16,), jnp.float32)
    zero16i = jnp.zeros((16,), jnp.int32)

    for i in range(K // 16):
        b_w[pl.ds(i * 16, 16)] = zero16f
        b_cols[pl.ds(i * 16, 16)] = zero16i
        b_rows[pl.ds(i * 16, 16)] = zero16i

    ebase = s * EPW

    def flush():
        pltpu.async_copy(y_hbm.at[b_cols], g_buf, sem).wait()

        def scale_body(k, carry):
            wspl = plsc.load_gather(b_w, [zero16i + k])
            for j in range(D // 32):
                xi = g_buf[k, pl.ds(j * 16, 16)]
                x = plsc.bitcast(xi, jnp.bfloat16)
                a, b = plsc.unpack(x, format=plsc.PackFormat.INTERLEAVED)
                s_buf[k, pl.ds(j * 32, 16)] = a * wspl
                s_buf[k, pl.ds(j * 32 + 16, 16)] = b * wspl
            return carry

        lax.fori_loop(0, K, scale_body, 0)
        pltpu.sync_copy(s_buf, acc.at[b_rows], add=True)
        for i in range(K // 16):
            b_w[pl.ds(i * 16, 16)] = zero16f

    for chunk in range(NCHUNK // 2):
        lo = (c * (NCHUNK // 2) + chunk) * CH
        abase = s * CPT

        def zero_sbuf(k, carry):
            for j in range(D // 16):
                s_buf[k, pl.ds(j * 16, 16)] = zero16f
            return carry

        lax.fori_loop(0, K, zero_sbuf, 0)
        nfull = CPT // K
        for t in range(nfull):
            pltpu.sync_copy(s_buf, acc.at[pl.ds(abase + t * K, K)])
        rem = CPT - nfull * K
        if rem:
            pltpu.sync_copy(s_buf.at[pl.ds(0, rem)],
                            acc.at[pl.ds(abase + nfull * K, rem)])
        plsc.subcore_barrier()

        def seg_body(sg, nbuf):
            off = ebase + sg * SEG
            pltpu.sync_copy(rows_hbm.at[pl.ds(off, SEG)], seg_rows)
            pltpu.sync_copy(cols_hbm.at[pl.ds(off, SEG)], seg_cols)
            pltpu.sync_copy(w_hbm.at[pl.ds(off, SEG)], seg_w)

            def scan_body(i, nb):
                o = i * 16
                rv = seg_rows[pl.ds(o, 16)]
                lrv = rv - lo
                m = lrv.astype(jnp.uint32) < jnp.uint32(CH)
                cv = seg_cols[pl.ds(o, 16)]
                wv = seg_w[pl.ds(o, 16)]
                mi = m.astype(jnp.int32)
                incl = plsc.cumsum(mi)
                idx = (incl - mi) + nb
                plsc.store_scatter(b_cols, [idx], cv, mask=m)
                plsc.store_scatter(b_rows, [idx], lrv, mask=m)
                plsc.store_scatter(b_w, [idx], wv, mask=m)
                cnt = plsc.all_reduce_population_count(m)
                nb2 = nb + cnt[0]

                @pl.when(nb2 >= THRESH)
                def _():
                    flush()

                return jnp.where(nb2 >= THRESH, 0, nb2)

            return lax.fori_loop(0, NVEC, scan_body, nbuf)

        lax.fori_loop(0, NSEG, seg_body, 0)
        flush()
        plsc.subcore_barrier()

        pltpu.sync_copy(acc.at[pl.ds(abase, CPT)],
                        out_hbm.at[pl.ds(lo + abase, CPT)])
        plsc.subcore_barrier()


def _w_body(rel_ref, W1_ref, b1_ref, W2t_ref, vals_ref, w_ref):
    h = jnp.tanh(jnp.dot(rel_ref[...], W1_ref[...],
                         preferred_element_type=jnp.float32) + b1_ref[...])
    logit = jnp.sum(h * W2t_ref[...], axis=1, keepdims=True)
    w_ref[...] = vals_ref[...] * jax.nn.sigmoid(logit)


def _mm_body(x_ref, w_ref, y_ref):
    y_ref[...] = jnp.dot(x_ref[...], w_ref[...],
                         preferred_element_type=jnp.float32).astype(jnp.bfloat16)


def kernel(ent_mat, rel_mat, adj_rows, adj_cols, adj_vals, weight_ent, W1, b1, W2):
    w2d = pl.pallas_call(
        _w_body,
        out_shape=jax.ShapeDtypeStruct((R, E), jnp.float32),
    )(rel_mat, W1, b1.reshape(1, D), W2.reshape(1, D), adj_vals)

    perm = _np.empty((D,), dtype=_np.int32)
    for g in range(D // 32):
        for i in range(16):
            perm[g * 32 + 2 * i] = g * 32 + i
            perm[g * 32 + 2 * i + 1] = g * 32 + 16 + i
    w_ent_p = weight_ent[:, perm]

    BM = 1000
    y = pl.pallas_call(
        _mm_body,
        grid=(N // BM,),
        in_specs=[pl.BlockSpec((BM, D), lambda i: (i, 0)),
                  pl.BlockSpec((D, D), lambda i: (0, 0))],
        out_specs=pl.BlockSpec((BM, D), lambda i: (i, 0)),
        out_shape=jax.ShapeDtypeStruct((N, D), jnp.bfloat16),
    )(ent_mat, w_ent_p)

    pad = NEP - NE
    rows_f = jnp.concatenate(
        [adj_rows.reshape(-1), jnp.full((pad,), 2 ** 30, jnp.int32)])
    cols_f = jnp.concatenate(
        [adj_cols.reshape(-1), jnp.zeros((pad,), jnp.int32)])
    w_f = jnp.concatenate([w2d.reshape(-1), jnp.zeros((pad,), jnp.float32)])

    y32 = jax.lax.bitcast_convert_type(y.reshape(N, D // 2, 2), jnp.int32)
    out_p = _sc_scatter(rows_f, cols_f, w_f, y32)
    return out_p[:N], rel_mat
